# initial kernel scaffold (unmeasured)
import jax
import jax.numpy as jnp
from jax import lax
from jax.experimental import pallas as pl
from jax.experimental.pallas import tpu as pltpu

N_DEV = 4
T = 4
MC = 1024


def kernel(x, w_mat, scale_x, scale_w):
    M, KL = x.shape
    _, N = w_mat.shape
    W = N // T
    WH = W // 2
    n_hops = 3 * T

    def body(x_ref, w_ref, sx_ref, sw_ref, out_ref,
             send_cw, send_ccw, recv_cw, recv_ccw, stage,
             send_sem_cw, send_sem_ccw, recv_sem_cw, recv_sem_ccw,
             credit_cw, credit_ccw, out_sem):
        p = lax.axis_index("i")
        right = lax.rem(p + 1, N_DEV)
        left = lax.rem(p + N_DEV - 1, N_DEV)
        scale = sx_ref[0] * sw_ref[0]

        def partial(c, col0, width):
            xs = x_ref[pl.ds(c * MC, MC), :]
            ws = w_ref[:, pl.ds(col0, width)]
            return jnp.dot(xs, ws, preferred_element_type=jnp.float32)

        def mk_rdma(send_buf, recv_buf, ssem, rsem, ss, rs, dst):
            return pltpu.make_async_remote_copy(
                src_ref=send_buf.at[ss],
                dst_ref=recv_buf.at[rs],
                send_sem=ssem.at[ss],
                recv_sem=rsem.at[rs],
                device_id=(dst,),
                device_id_type=pl.DeviceIdType.MESH,
            )

        out_copy = None
        for t in range(T):
            base = t * W
            carry_cw = None
            carry_ccw = None
            for s in range(3):
                j = 3 * t + s
                ss = j % 2
                rs = j % 3
                c_cw = lax.rem(p + (3 - s), N_DEV)
                c_ccw = lax.rem(p + s + 1, N_DEV)
                val_cw = partial(c_cw, base, WH)
                val_ccw = partial(c_ccw, base + WH, WH)
                if carry_cw is not None:
                    val_cw = val_cw + carry_cw
                    val_ccw = val_ccw + carry_ccw
                if j >= 3:
                    pl.semaphore_wait(credit_cw, 1)
                    pl.semaphore_wait(credit_ccw, 1)
                send_cw[ss] = val_cw.astype(jnp.bfloat16)
                send_ccw[ss] = val_ccw.astype(jnp.bfloat16)
                rd_cw = mk_rdma(send_cw, recv_cw, send_sem_cw,
                                recv_sem_cw, ss, rs, right)
                rd_ccw = mk_rdma(send_ccw, recv_ccw, send_sem_ccw,
                                 recv_sem_ccw, ss, rs, left)
                rd_cw.start()
                rd_ccw.start()
                rd_cw.wait()
                rd_ccw.wait()
                carry_cw = recv_cw[rs].astype(jnp.float32)
                carry_ccw = recv_ccw[rs].astype(jnp.float32)
                if j <= n_hops - 4:
                    pl.semaphore_signal(
                        credit_cw, 1, device_id=(left,),
                        device_id_type=pl.DeviceIdType.MESH)
                    pl.semaphore_signal(
                        credit_ccw, 1, device_id=(right,),
                        device_id_type=pl.DeviceIdType.MESH)
            pfull = partial(p, base, W)
            acc_cw = pfull[:, :WH] + carry_cw
            acc_ccw = pfull[:, WH:] + carry_ccw
            if out_copy is not None:
                out_copy.wait()
            stage[:, :WH] = jnp.maximum(acc_cw * scale, 0.0)
            stage[:, WH:] = jnp.maximum(acc_ccw * scale, 0.0)
            out_copy = pltpu.make_async_copy(
                stage, out_ref.at[:, pl.ds(base, W)], out_sem)
            out_copy.start()
        out_copy.wait()

    return pl.pallas_call(
        body,
        out_shape=jax.ShapeDtypeStruct((MC, N), jnp.float32),
        in_specs=[
            pl.BlockSpec(memory_space=pltpu.VMEM),
            pl.BlockSpec(memory_space=pltpu.VMEM),
            pl.BlockSpec(memory_space=pltpu.SMEM),
            pl.BlockSpec(memory_space=pltpu.SMEM),
        ],
        out_specs=pl.BlockSpec(memory_space=pl.ANY),
        scratch_shapes=[
            pltpu.VMEM((2, MC, WH), jnp.bfloat16),
            pltpu.VMEM((2, MC, WH), jnp.bfloat16),
            pltpu.VMEM((3, MC, WH), jnp.bfloat16),
            pltpu.VMEM((3, MC, WH), jnp.bfloat16),
            pltpu.VMEM((MC, W), jnp.float32),
            pltpu.SemaphoreType.DMA((2,)),
            pltpu.SemaphoreType.DMA((2,)),
            pltpu.SemaphoreType.DMA((3,)),
            pltpu.SemaphoreType.DMA((3,)),
            pltpu.SemaphoreType.REGULAR,
            pltpu.SemaphoreType.REGULAR,
            pltpu.SemaphoreType.DMA,
        ],
    )(x, w_mat, scale_x, scale_w)


# baseline (device time: 403518 ns/iter reference)
import jax
import jax.numpy as jnp
from jax import lax
from jax.experimental import pallas as pl
from jax.experimental.pallas import tpu as pltpu

N_DEV = 4
T = 4
MC = 1024


def kernel(x, w_mat, scale_x, scale_w):
    x = x.astype(jnp.float8_e4m3fn)
    w_mat = w_mat.astype(jnp.float8_e4m3fn)
    M, KL = x.shape
    _, N = w_mat.shape
    W = N // T
    WH = W // 2
    n_hops = 3 * T

    def body(x_ref, w_ref, sx_ref, sw_ref, out_ref,
             send_cw, send_ccw, recv_cw, recv_ccw, stage,
             send_sem_cw, send_sem_ccw, recv_sem_cw, recv_sem_ccw,
             credit_cw, credit_ccw, out_sem):
        p = lax.axis_index("i")
        right = lax.rem(p + 1, N_DEV)
        left = lax.rem(p + N_DEV - 1, N_DEV)
        scale = sx_ref[0] * sw_ref[0]

        def partial(c, col0, width):
            xs = x_ref[pl.ds(c * MC, MC), :]
            ws = w_ref[:, pl.ds(col0, width)]
            return jnp.dot(xs, ws, preferred_element_type=jnp.float32)

        def mk_rdma(send_buf, recv_buf, ssem, rsem, ss, rs, dst):
            return pltpu.make_async_remote_copy(
                src_ref=send_buf.at[ss],
                dst_ref=recv_buf.at[rs],
                send_sem=ssem.at[ss],
                recv_sem=rsem.at[rs],
                device_id=(dst,),
                device_id_type=pl.DeviceIdType.MESH,
            )

        out_copy = None
        for t in range(T):
            base = t * W
            carry_cw = None
            carry_ccw = None
            for s in range(3):
                j = 3 * t + s
                ss = j % 2
                rs = j % 3
                c_cw = lax.rem(p + (3 - s), N_DEV)
                c_ccw = lax.rem(p + s + 1, N_DEV)
                val_cw = partial(c_cw, base, WH)
                val_ccw = partial(c_ccw, base + WH, WH)
                if carry_cw is not None:
                    val_cw = val_cw + carry_cw
                    val_ccw = val_ccw + carry_ccw
                if j >= 3:
                    pl.semaphore_wait(credit_cw, 1)
                    pl.semaphore_wait(credit_ccw, 1)
                send_cw[ss] = val_cw.astype(jnp.bfloat16)
                send_ccw[ss] = val_ccw.astype(jnp.bfloat16)
                rd_cw = mk_rdma(send_cw, recv_cw, send_sem_cw,
                                recv_sem_cw, ss, rs, right)
                rd_ccw = mk_rdma(send_ccw, recv_ccw, send_sem_ccw,
                                 recv_sem_ccw, ss, rs, left)
                rd_cw.start()
                rd_ccw.start()
                rd_cw.wait()
                rd_ccw.wait()
                carry_cw = recv_cw[rs].astype(jnp.float32)
                carry_ccw = recv_ccw[rs].astype(jnp.float32)
                if j <= n_hops - 4:
                    pl.semaphore_signal(
                        credit_cw, 1, device_id=(left,),
                        device_id_type=pl.DeviceIdType.MESH)
                    pl.semaphore_signal(
                        credit_ccw, 1, device_id=(right,),
                        device_id_type=pl.DeviceIdType.MESH)
            pfull = partial(p, base, W)
            acc_cw = pfull[:, :WH] + carry_cw
            acc_ccw = pfull[:, WH:] + carry_ccw
            if out_copy is not None:
                out_copy.wait()
            stage[:, :WH] = jnp.maximum(acc_cw * scale, 0.0)
            stage[:, WH:] = jnp.maximum(acc_ccw * scale, 0.0)
            out_copy = pltpu.make_async_copy(
                stage, out_ref.at[:, pl.ds(base, W)], out_sem)
            out_copy.start()
        out_copy.wait()

    return pl.pallas_call(
        body,
        out_shape=jax.ShapeDtypeStruct((MC, N), jnp.float32),
        in_specs=[
            pl.BlockSpec(memory_space=pltpu.VMEM),
            pl.BlockSpec(memory_space=pltpu.VMEM),
            pl.BlockSpec(memory_space=pltpu.SMEM),
            pl.BlockSpec(memory_space=pltpu.SMEM),
        ],
        out_specs=pl.BlockSpec(memory_space=pltpu.HBM),
        compiler_params=pltpu.CompilerParams(
            vmem_limit_bytes=62 * 1024 * 1024),
        scratch_shapes=[
            pltpu.VMEM((2, MC, WH), jnp.bfloat16),
            pltpu.VMEM((2, MC, WH), jnp.bfloat16),
            pltpu.VMEM((3, MC, WH), jnp.bfloat16),
            pltpu.VMEM((3, MC, WH), jnp.bfloat16),
            pltpu.VMEM((MC, W), jnp.float32),
            pltpu.SemaphoreType.DMA((2,)),
            pltpu.SemaphoreType.DMA((2,)),
            pltpu.SemaphoreType.DMA((3,)),
            pltpu.SemaphoreType.DMA((3,)),
            pltpu.SemaphoreType.REGULAR,
            pltpu.SemaphoreType.REGULAR,
            pltpu.SemaphoreType.DMA,
        ],
    )(x, w_mat, scale_x, scale_w)


# device time: 334940 ns/iter; 1.2047x vs baseline; 1.2047x over previous
import jax
import jax.numpy as jnp
from jax import lax
from jax.experimental import pallas as pl
from jax.experimental.pallas import tpu as pltpu

N_DEV = 4
T = 4
MC = 1024
NRS = 4
NSS = 2


def _schedule():
    sched = []
    for q in range(0, T, 2):
        a, b = q, q + 1
        if q == 0:
            for s in range(3):
                sched += [(a, s), (b, s)]
        else:
            sched += [(a, 0), (b, 0), (a - 2, "f"), (b - 2, "f")]
            for s in range(1, 3):
                sched += [(a, s), (b, s)]
    sched += [(T - 2, "f"), (T - 1, "f")]
    return sched


def kernel(x, w_mat, scale_x, scale_w):
    x = x.astype(jnp.float8_e4m3fn)
    w_mat = w_mat.astype(jnp.float8_e4m3fn)
    M, KL = x.shape
    _, N = w_mat.shape
    W = N // T
    WH = W // 2
    K_SENDS = 3 * T

    sched = _schedule()
    send_idx = {}
    k = 0
    for (t, s) in sched:
        if s != "f":
            send_idx[(t, s)] = k
            k += 1

    def body(x_ref, w_ref, sx_ref, sw_ref, out_ref,
             send_cw, send_ccw, recv_cw, recv_ccw, stage,
             send_sem_cw, send_sem_ccw, recv_sem_cw, recv_sem_ccw,
             credit_cw, credit_ccw, out_sem):
        p = lax.axis_index("i")
        right = lax.rem(p + 1, N_DEV)
        left = lax.rem(p + N_DEV - 1, N_DEV)
        scale = sx_ref[0] * sw_ref[0]

        def partial(c, col0, width):
            xs = x_ref[pl.ds(c * MC, MC), :]
            ws = w_ref[:, pl.ds(col0, width)]
            return jnp.dot(xs, ws, preferred_element_type=jnp.float32)

        def mk_rdma(d, ss, rs, dst):
            send_buf, recv_buf = (send_cw, recv_cw) if d == 0 else (send_ccw, recv_ccw)
            ssem, rsem = (send_sem_cw, recv_sem_cw) if d == 0 else (send_sem_ccw, recv_sem_ccw)
            return pltpu.make_async_remote_copy(
                src_ref=send_buf.at[ss],
                dst_ref=recv_buf.at[rs],
                send_sem=ssem.at[ss],
                recv_sem=rsem.at[rs],
                device_id=(dst,),
                device_id_type=pl.DeviceIdType.MESH,
            )

        def consume(t, s_prev):
            kp = send_idx[(t, s_prev)]
            rp = kp % NRS
            rd_in_cw = mk_rdma(0, kp % NSS, rp, right)
            rd_in_ccw = mk_rdma(1, kp % NSS, rp, left)
            rd_in_cw.wait_recv()
            rd_in_ccw.wait_recv()
            got_cw = recv_cw[rp].astype(jnp.float32)
            got_ccw = recv_ccw[rp].astype(jnp.float32)
            if kp <= K_SENDS - 1 - NRS:
                pl.semaphore_signal(credit_cw, 1, device_id=(left,),
                                    device_id_type=pl.DeviceIdType.MESH)
                pl.semaphore_signal(credit_ccw, 1, device_id=(right,),
                                    device_id_type=pl.DeviceIdType.MESH)
            return got_cw, got_ccw

        pend = {}
        out_copy = None
        for (t, s) in sched:
            base = t * W
            if s == "f":
                pfull = partial(p, base, W)
                got_cw, got_ccw = consume(t, 2)
                acc_cw = pfull[:, :WH] + got_cw
                acc_ccw = pfull[:, WH:] + got_ccw
                if out_copy is not None:
                    out_copy.wait()
                stage[:, :WH] = jnp.maximum(acc_cw * scale, 0.0)
                stage[:, WH:] = jnp.maximum(acc_ccw * scale, 0.0)
                out_copy = pltpu.make_async_copy(
                    stage, out_ref.at[:, pl.ds(base, W)], out_sem)
                out_copy.start()
                continue
            kk = send_idx[(t, s)]
            ss = kk % NSS
            rs = kk % NRS
            c_cw = lax.rem(p + (3 - s), N_DEV)
            c_ccw = lax.rem(p + s + 1, N_DEV)
            val_cw = partial(c_cw, base, WH)
            val_ccw = partial(c_ccw, base + WH, WH)
            if s > 0:
                got_cw, got_ccw = consume(t, s - 1)
                val_cw = val_cw + got_cw
                val_ccw = val_ccw + got_ccw
            if kk >= NRS:
                pl.semaphore_wait(credit_cw, 1)
                pl.semaphore_wait(credit_ccw, 1)
            if (0, ss) in pend:
                pend[(0, ss)].wait_send()
                pend[(1, ss)].wait_send()
            send_cw[ss] = val_cw.astype(jnp.bfloat16)
            send_ccw[ss] = val_ccw.astype(jnp.bfloat16)
            rd_cw = mk_rdma(0, ss, rs, right)
            rd_ccw = mk_rdma(1, ss, rs, left)
            rd_cw.start()
            rd_ccw.start()
            pend[(0, ss)] = rd_cw
            pend[(1, ss)] = rd_ccw
        for ss in range(NSS):
            pend[(0, ss)].wait_send()
            pend[(1, ss)].wait_send()
        out_copy.wait()

    return pl.pallas_call(
        body,
        out_shape=jax.ShapeDtypeStruct((MC, N), jnp.float32),
        in_specs=[
            pl.BlockSpec(memory_space=pltpu.VMEM),
            pl.BlockSpec(memory_space=pltpu.VMEM),
            pl.BlockSpec(memory_space=pltpu.SMEM),
            pl.BlockSpec(memory_space=pltpu.SMEM),
        ],
        out_specs=pl.BlockSpec(memory_space=pltpu.HBM),
        compiler_params=pltpu.CompilerParams(
            vmem_limit_bytes=67000000),
        scratch_shapes=[
            pltpu.VMEM((NSS, MC, WH), jnp.bfloat16),
            pltpu.VMEM((NSS, MC, WH), jnp.bfloat16),
            pltpu.VMEM((NRS, MC, WH), jnp.bfloat16),
            pltpu.VMEM((NRS, MC, WH), jnp.bfloat16),
            pltpu.VMEM((MC, W), jnp.float32),
            pltpu.SemaphoreType.DMA((NSS,)),
            pltpu.SemaphoreType.DMA((NSS,)),
            pltpu.SemaphoreType.DMA((NRS,)),
            pltpu.SemaphoreType.DMA((NRS,)),
            pltpu.SemaphoreType.REGULAR,
            pltpu.SemaphoreType.REGULAR,
            pltpu.SemaphoreType.DMA,
        ],
    )(x, w_mat, scale_x, scale_w)


# device time: 321619 ns/iter; 1.2546x vs baseline; 1.0414x over previous
import jax
import jax.numpy as jnp
from jax import lax
from jax.experimental import pallas as pl
from jax.experimental.pallas import tpu as pltpu

N_DEV = 4
T = 4
MC = 1024
NRS = 4
NSS = 2


def _schedule():
    sched = []
    for q in range(0, T, 2):
        a, b = q, q + 1
        if q == 0:
            for s in range(3):
                sched += [(a, s), (b, s)]
        else:
            sched += [(a, 0), (b, 0), (a - 2, "f"), (b - 2, "f")]
            for s in range(1, 3):
                sched += [(a, s), (b, s)]
    sched += [(T - 2, "f"), (T - 1, "f")]
    return sched


def kernel(x, w_mat, scale_x, scale_w):
    x = x.astype(jnp.float8_e4m3fn)
    M, KL = x.shape
    _, N = w_mat.shape
    W = N // T
    WH = W // 2
    K_SENDS = 3 * T

    sched = _schedule()
    send_idx = {}
    k = 0
    for (t, s) in sched:
        if s != "f":
            send_idx[(t, s)] = k
            k += 1

    def body(x_ref, w_hbm, sx_ref, sw_ref, out_ref,
             w8, wstg, send_cw, send_ccw, recv_cw, recv_ccw,
             send_sem_cw, send_sem_ccw, recv_sem_cw, recv_sem_ccw,
             credit_cw, credit_ccw, wdma_sem, out_sem):
        p = lax.axis_index("i")
        right = lax.rem(p + 1, N_DEV)
        left = lax.rem(p + N_DEV - 1, N_DEV)
        scale = sx_ref[0] * sw_ref[0]
        stage = wstg

        def w_dma(t):
            return pltpu.make_async_copy(
                w_hbm.at[:, pl.ds(t * W, W)], wstg, wdma_sem)

        def partial(c, col0, width):
            xs = x_ref[pl.ds(c * MC, MC), :]
            ws = w8[:, pl.ds(col0, width)]
            return jnp.dot(xs, ws, preferred_element_type=jnp.float32)

        def mk_rdma(d, ss, rs, dst):
            send_buf, recv_buf = (send_cw, recv_cw) if d == 0 else (send_ccw, recv_ccw)
            ssem, rsem = (send_sem_cw, recv_sem_cw) if d == 0 else (send_sem_ccw, recv_sem_ccw)
            return pltpu.make_async_remote_copy(
                src_ref=send_buf.at[ss],
                dst_ref=recv_buf.at[rs],
                send_sem=ssem.at[ss],
                recv_sem=rsem.at[rs],
                device_id=(dst,),
                device_id_type=pl.DeviceIdType.MESH,
            )

        def consume(t, s_prev):
            kp = send_idx[(t, s_prev)]
            rp = kp % NRS
            rd_in_cw = mk_rdma(0, kp % NSS, rp, right)
            rd_in_ccw = mk_rdma(1, kp % NSS, rp, left)
            rd_in_cw.wait_recv()
            rd_in_ccw.wait_recv()
            got_cw = recv_cw[rp].astype(jnp.float32)
            got_ccw = recv_ccw[rp].astype(jnp.float32)
            if kp <= K_SENDS - 1 - NRS:
                pl.semaphore_signal(credit_cw, 1, device_id=(left,),
                                    device_id_type=pl.DeviceIdType.MESH)
                pl.semaphore_signal(credit_ccw, 1, device_id=(right,),
                                    device_id_type=pl.DeviceIdType.MESH)
            return got_cw, got_ccw

        pend = {}
        out_copy = None
        w_dma(0).start()
        for (t, s) in sched:
            base = t * W
            if s == 0:
                w_dma(t).wait()
                w8[:, pl.ds(base, W)] = wstg[:, :].astype(jnp.float8_e4m3fn)
                if t + 1 < T:
                    w_dma(t + 1).start()
            if s == "f":
                pfull = partial(p, base, W)
                got_cw, got_ccw = consume(t, 2)
                acc_cw = pfull[:, :WH] + got_cw
                acc_ccw = pfull[:, WH:] + got_ccw
                if out_copy is not None:
                    out_copy.wait()
                stage[:, :WH] = jnp.maximum(acc_cw * scale, 0.0)
                stage[:, WH:] = jnp.maximum(acc_ccw * scale, 0.0)
                out_copy = pltpu.make_async_copy(
                    stage, out_ref.at[:, pl.ds(base, W)], out_sem)
                out_copy.start()
                continue
            kk = send_idx[(t, s)]
            ss = kk % NSS
            rs = kk % NRS
            c_cw = lax.rem(p + (3 - s), N_DEV)
            c_ccw = lax.rem(p + s + 1, N_DEV)
            val_cw = partial(c_cw, base, WH)
            val_ccw = partial(c_ccw, base + WH, WH)
            if s > 0:
                got_cw, got_ccw = consume(t, s - 1)
                val_cw = val_cw + got_cw
                val_ccw = val_ccw + got_ccw
            if kk >= NRS:
                pl.semaphore_wait(credit_cw, 1)
                pl.semaphore_wait(credit_ccw, 1)
            if (0, ss) in pend:
                pend[(0, ss)].wait_send()
                pend[(1, ss)].wait_send()
            send_cw[ss] = val_cw.astype(jnp.bfloat16)
            send_ccw[ss] = val_ccw.astype(jnp.bfloat16)
            rd_cw = mk_rdma(0, ss, rs, right)
            rd_ccw = mk_rdma(1, ss, rs, left)
            rd_cw.start()
            rd_ccw.start()
            pend[(0, ss)] = rd_cw
            pend[(1, ss)] = rd_ccw
        for ss in range(NSS):
            pend[(0, ss)].wait_send()
            pend[(1, ss)].wait_send()
        out_copy.wait()

    return pl.pallas_call(
        body,
        out_shape=jax.ShapeDtypeStruct((MC, N), jnp.float32),
        in_specs=[
            pl.BlockSpec(memory_space=pltpu.VMEM),
            pl.BlockSpec(memory_space=pltpu.HBM),
            pl.BlockSpec(memory_space=pltpu.SMEM),
            pl.BlockSpec(memory_space=pltpu.SMEM),
        ],
        out_specs=pl.BlockSpec(memory_space=pltpu.HBM),
        compiler_params=pltpu.CompilerParams(
            vmem_limit_bytes=67000000),
        scratch_shapes=[
            pltpu.VMEM((KL, N), jnp.float8_e4m3fn),
            pltpu.VMEM((KL, W), jnp.float32),
            pltpu.VMEM((NSS, MC, WH), jnp.bfloat16),
            pltpu.VMEM((NSS, MC, WH), jnp.bfloat16),
            pltpu.VMEM((NRS, MC, WH), jnp.bfloat16),
            pltpu.VMEM((NRS, MC, WH), jnp.bfloat16),
            pltpu.SemaphoreType.DMA((NSS,)),
            pltpu.SemaphoreType.DMA((NSS,)),
            pltpu.SemaphoreType.DMA((NRS,)),
            pltpu.SemaphoreType.DMA((NRS,)),
            pltpu.SemaphoreType.REGULAR,
            pltpu.SemaphoreType.REGULAR,
            pltpu.SemaphoreType.DMA,
            pltpu.SemaphoreType.DMA,
        ],
    )(x, w_mat, scale_x, scale_w)


# device time: 314655 ns/iter; 1.2824x vs baseline; 1.0221x over previous
import jax
import jax.numpy as jnp
from jax import lax
from jax.experimental import pallas as pl
from jax.experimental.pallas import tpu as pltpu

N_DEV = 4
T = 4
MC = 1024
NRS = 4
NSS = 2


def _schedule():
    sched = []
    for q in range(0, T, 2):
        a, b = q, q + 1
        if q == 0:
            for s in range(3):
                sched += [(a, s), (b, s)]
        else:
            sched += [(a, 0), (b, 0), (a - 2, "f"), (b - 2, "f")]
            for s in range(1, 3):
                sched += [(a, s), (b, s)]
    sched += [(T - 2, "f"), (T - 1, "f")]
    return sched


def kernel(x, w_mat, scale_x, scale_w):
    M, KL = x.shape
    _, N = w_mat.shape
    W = N // T
    WH = W // 2
    K_SENDS = 3 * T

    sched = _schedule()
    send_idx = {}
    k = 0
    for (t, s) in sched:
        if s != "f":
            send_idx[(t, s)] = k
            k += 1

    def body(x_hbm, w_hbm, sx_ref, sw_ref, out_ref,
             x8, xstg, w8, wstg, send_cw, send_ccw, recv_cw, recv_ccw,
             send_sem_cw, send_sem_ccw, recv_sem_cw, recv_sem_ccw,
             credit_cw, credit_ccw, xdma_sem, wdma_sem, out_sem):
        p = lax.axis_index("i")
        right = lax.rem(p + 1, N_DEV)
        left = lax.rem(p + N_DEV - 1, N_DEV)
        scale = sx_ref[0] * sw_ref[0]
        stage = wstg

        def w_dma(t):
            return pltpu.make_async_copy(
                w_hbm.at[:, pl.ds(t * W, W)], wstg, wdma_sem)

        def x_dma(c):
            return pltpu.make_async_copy(
                x_hbm.at[pl.ds(c * MC, MC), :], xstg, xdma_sem)

        x_order = [lax.rem(p + 3, N_DEV), lax.rem(p + 1, N_DEV),
                   lax.rem(p + 2, N_DEV), p]

        def x_convert(i):
            c = x_order[i]
            x_dma(c).wait()
            x8[pl.ds(c * MC, MC), :] = xstg[:, :].astype(jnp.float8_e4m3fn)
            if i + 1 < len(x_order):
                x_dma(x_order[i + 1]).start()

        def partial(c, col0, width):
            xs = x8[pl.ds(c * MC, MC), :]
            ws = w8[:, pl.ds(col0, width)]
            return jnp.dot(xs, ws, preferred_element_type=jnp.float32)

        def mk_rdma(d, ss, rs, dst):
            send_buf, recv_buf = (send_cw, recv_cw) if d == 0 else (send_ccw, recv_ccw)
            ssem, rsem = (send_sem_cw, recv_sem_cw) if d == 0 else (send_sem_ccw, recv_sem_ccw)
            return pltpu.make_async_remote_copy(
                src_ref=send_buf.at[ss],
                dst_ref=recv_buf.at[rs],
                send_sem=ssem.at[ss],
                recv_sem=rsem.at[rs],
                device_id=(dst,),
                device_id_type=pl.DeviceIdType.MESH,
            )

        def consume(t, s_prev):
            kp = send_idx[(t, s_prev)]
            rp = kp % NRS
            rd_in_cw = mk_rdma(0, kp % NSS, rp, right)
            rd_in_ccw = mk_rdma(1, kp % NSS, rp, left)
            rd_in_cw.wait_recv()
            rd_in_ccw.wait_recv()
            got_cw = recv_cw[rp].astype(jnp.float32)
            got_ccw = recv_ccw[rp].astype(jnp.float32)
            if kp <= K_SENDS - 1 - NRS:
                pl.semaphore_signal(credit_cw, 1, device_id=(left,),
                                    device_id_type=pl.DeviceIdType.MESH)
                pl.semaphore_signal(credit_ccw, 1, device_id=(right,),
                                    device_id_type=pl.DeviceIdType.MESH)
            return got_cw, got_ccw

        pend = {}
        out_copy = None
        x_dma(x_order[0]).start()
        w_dma(0).start()
        x_convert(0)
        x_convert(1)
        for pos, (t, s) in enumerate(sched):
            base = t * W
            if s == 0:
                w_dma(t).wait()
                w8[:, pl.ds(base, W)] = wstg[:, :].astype(jnp.float8_e4m3fn)
                if t + 1 < T:
                    w_dma(t + 1).start()
            if s == "f":
                pfull = partial(p, base, W)
                got_cw, got_ccw = consume(t, 2)
                acc_cw = pfull[:, :WH] + got_cw
                acc_ccw = pfull[:, WH:] + got_ccw
                if out_copy is not None:
                    out_copy.wait()
                stage[:, :WH] = jnp.maximum(acc_cw * scale, 0.0)
                stage[:, WH:] = jnp.maximum(acc_ccw * scale, 0.0)
                out_copy = pltpu.make_async_copy(
                    stage, out_ref.at[:, pl.ds(base, W)], out_sem)
                out_copy.start()
                continue
            kk = send_idx[(t, s)]
            ss = kk % NSS
            rs = kk % NRS
            c_cw = lax.rem(p + (3 - s), N_DEV)
            c_ccw = lax.rem(p + s + 1, N_DEV)
            val_cw = partial(c_cw, base, WH)
            val_ccw = partial(c_ccw, base + WH, WH)
            if s > 0:
                got_cw, got_ccw = consume(t, s - 1)
                val_cw = val_cw + got_cw
                val_ccw = val_ccw + got_ccw
            if kk >= NRS:
                pl.semaphore_wait(credit_cw, 1)
                pl.semaphore_wait(credit_ccw, 1)
            if (0, ss) in pend:
                pend[(0, ss)].wait_send()
                pend[(1, ss)].wait_send()
            send_cw[ss] = val_cw.astype(jnp.bfloat16)
            send_ccw[ss] = val_ccw.astype(jnp.bfloat16)
            rd_cw = mk_rdma(0, ss, rs, right)
            rd_ccw = mk_rdma(1, ss, rs, left)
            rd_cw.start()
            rd_ccw.start()
            pend[(0, ss)] = rd_cw
            pend[(1, ss)] = rd_ccw
            if pos == 0:
                x_convert(2)
            elif pos == 1:
                x_convert(3)
        for ss in range(NSS):
            pend[(0, ss)].wait_send()
            pend[(1, ss)].wait_send()
        out_copy.wait()

    return pl.pallas_call(
        body,
        out_shape=jax.ShapeDtypeStruct((MC, N), jnp.float32),
        in_specs=[
            pl.BlockSpec(memory_space=pltpu.HBM),
            pl.BlockSpec(memory_space=pltpu.HBM),
            pl.BlockSpec(memory_space=pltpu.SMEM),
            pl.BlockSpec(memory_space=pltpu.SMEM),
        ],
        out_specs=pl.BlockSpec(memory_space=pltpu.HBM),
        compiler_params=pltpu.CompilerParams(
            vmem_limit_bytes=67000000),
        scratch_shapes=[
            pltpu.VMEM((M, KL), jnp.float8_e4m3fn),
            pltpu.VMEM((MC, KL), jnp.float32),
            pltpu.VMEM((KL, N), jnp.float8_e4m3fn),
            pltpu.VMEM((KL, W), jnp.float32),
            pltpu.VMEM((NSS, MC, WH), jnp.bfloat16),
            pltpu.VMEM((NSS, MC, WH), jnp.bfloat16),
            pltpu.VMEM((NRS, MC, WH), jnp.bfloat16),
            pltpu.VMEM((NRS, MC, WH), jnp.bfloat16),
            pltpu.SemaphoreType.DMA((NSS,)),
            pltpu.SemaphoreType.DMA((NSS,)),
            pltpu.SemaphoreType.DMA((NRS,)),
            pltpu.SemaphoreType.DMA((NRS,)),
            pltpu.SemaphoreType.REGULAR,
            pltpu.SemaphoreType.REGULAR,
            pltpu.SemaphoreType.DMA,
            pltpu.SemaphoreType.DMA,
            pltpu.SemaphoreType.DMA,
        ],
    )(x, w_mat, scale_x, scale_w)
